# baseline (speedup 1.0000x reference)
import jax
import jax.numpy as jnp
from jax.experimental import pallas as pl

_BLK = 2000
_P = 64

_W10, _W12, _W13 = 0, 128, 256
_W20, _W22, _W23 = 384, 448, 512
_B10, _B12, _B13, _WC1 = 576, 584, 592, 600
_B20, _B22, _B23, _WC2 = 608, 616, 624, 632
_LINW, _LINB = 640, 704
_ROWS = 712


def _fused_kernel(x_ref, p_ref, out_ref):
    def mm(a, w):
        return jnp.dot(a, w, preferred_element_type=jnp.float32)

    def row(r):
        return p_ref[r:r + 1, :]

    def cell(h, w0r, w2r, w3r, nrows, b0r, b2r, b3r, wcr):
        w0 = p_ref[w0r:w0r + nrows, :]
        w2 = p_ref[w2r:w2r + nrows, :]
        w3 = p_ref[w3r:w3r + nrows, :]
        i = 0.5 * jnp.tanh(mm(h, w0) + row(b0r)) + 0.5
        t = jnp.tanh(mm(h, w2) + row(b2r))
        c = i * t
        o = 0.5 * jnp.tanh(mm(h, w3) + row(b3r) + row(wcr) * c) + 0.5
        return o * jnp.tanh(c)

    h = cell(x_ref[...], _W10, _W12, _W13, 128, _B10, _B12, _B13, _WC1)
    h = cell(h, _W20, _W22, _W23, _P, _B20, _B22, _B23, _WC2)
    h = jnp.maximum(h, 0.0)
    out_ref[...] = (mm(h, p_ref[_LINW:_LINW + _P, 0:1])
                    + p_ref[_LINB:_LINB + 1, 0:1])


def kernel(x, edge_index, edge_weight, l1_W, l1_b, l1_T0, l1_T1, l1_cb, l1_wc,
           l2_W, l2_b, l2_T0, l2_T1, l2_cb, l2_wc, lin_W, lin_b):
    n, d_in = x.shape

    def padc(a):
        return jnp.pad(a, ((0, 0), (0, _P - a.shape[1])))

    def block(a, rows):
        return jnp.pad(padc(a), ((0, rows - a.shape[0]), (0, 0)))

    parts = [
        block(0.5 * l1_W[0], 128), block(l1_W[2], 128), block(0.5 * l1_W[3], 128),
        block(0.5 * l2_W[0], _P), block(l2_W[2], _P), block(0.5 * l2_W[3], _P),
        block(0.5 * (l1_b[0] + l1_cb[0][None, :]), 8),
        block(l1_b[2] + l1_cb[2][None, :], 8),
        block(0.5 * (l1_b[3] + l1_cb[3][None, :]), 8),
        block(0.5 * l1_wc[2], 8),
        block(0.5 * (l2_b[0] + l2_cb[0][None, :]), 8),
        block(l2_b[2] + l2_cb[2][None, :], 8),
        block(0.5 * (l2_b[3] + l2_cb[3][None, :]), 8),
        block(0.5 * l2_wc[2], 8),
        block(lin_W, _P),
        block(lin_b.reshape(1, 1), 8),
    ]
    params = jnp.concatenate(parts, axis=0)

    return pl.pallas_call(
        _fused_kernel,
        grid=(n // _BLK,),
        in_specs=[
            pl.BlockSpec((_BLK, d_in), lambda i: (i, 0)),
            pl.BlockSpec((_ROWS, _P), lambda i: (0, 0)),
        ],
        out_specs=pl.BlockSpec((_BLK, 1), lambda i: (i, 0)),
        out_shape=jax.ShapeDtypeStruct((n, 1), jnp.float32),
    )(x, params)
